# baseline (speedup 1.0000x reference)
import functools

import jax
import jax.numpy as jnp
from jax.experimental import pallas as pl
from jax.experimental.pallas import tpu as pltpu

LANE = 128
SUBLANE = 8


def _ru(x, m):
    return ((x + m - 1) // m) * m


def _fused_kernel(len_ref, ehr_ref, imgT_ref, w_ehr_ref, w_cxr_ref,
                  w_proj_ref, b_proj_ref, w_cls_ref, b_cls_ref,
                  preds_ref, proj_ref, ehrf_ref, *, TP, FP, HW, HWP, C, DP):
    TB = len_ref.shape[0]
    lens = len_ref[...]

    ti = jax.lax.broadcasted_iota(jnp.int32, (TB, TP, 1), 1)
    mask3 = (lens[:, :, None] > ti).astype(jnp.float32)
    acc3 = jnp.zeros((TB, SUBLANE, FP), dtype=jnp.float32)
    for j in range(TP // SUBLANE):
        sl = pl.ds(j * SUBLANE, SUBLANE)
        acc3 = acc3 + ehr_ref[:, sl, :] * mask3[:, j * SUBLANE:(j + 1) * SUBLANE, :]
    acc = jnp.sum(acc3, axis=1)

    inv_len = 1.0 / jnp.maximum(lens.astype(jnp.float32), 1.0)
    ehr_feats = jnp.tanh(
        jnp.dot(acc, w_ehr_ref[...], preferred_element_type=jnp.float32) * inv_len)
    ehrf_ref[...] = ehr_feats

    inv_hw = jnp.float32(1.0 / HW)
    gparts = [jnp.sum(imgT_ref[c * HWP:(c + 1) * HWP, :], axis=0, keepdims=True)
              for c in range(C)]
    g3 = jnp.concatenate(gparts, axis=0) * inv_hw
    cxrT = jax.nn.relu(
        jax.lax.dot_general(w_cxr_ref[...], g3,
                            dimension_numbers=(((0,), (0,)), ((), ())),
                            preferred_element_type=jnp.float32))
    projT = (jax.lax.dot_general(w_proj_ref[...], cxrT,
                                 dimension_numbers=(((0,), (0,)), ((), ())),
                                 preferred_element_type=jnp.float32)
             + jnp.transpose(b_proj_ref[...]))

    proj_ref[...] = jnp.transpose(projT)

    logits = (jnp.dot(ehr_feats, w_cls_ref[0:DP, :], preferred_element_type=jnp.float32)
              + jax.lax.dot_general(projT, w_cls_ref[DP:2 * DP, :],
                                    dimension_numbers=(((0,), (0,)), ((), ())),
                                    preferred_element_type=jnp.float32)
              + b_cls_ref[...])
    preds_ref[...] = jax.nn.sigmoid(logits)


def kernel(ehr, ehr_len, img, w_ehr, w_cxr, w_proj, b_proj, w_cls, b_cls):
    B, T, F = ehr.shape
    C, H, W = img.shape[1], img.shape[2], img.shape[3]
    R, D = w_proj.shape
    C_out = b_cls.shape[1]
    HW = H * W

    FP = _ru(F, LANE)
    DP = _ru(D, LANE)
    RP = _ru(R, LANE)
    CP = _ru(C_out, LANE)
    HWP = _ru(HW, LANE)
    TP = _ru(T, SUBLANE)

    if (F, D) != (FP, DP):
        w_ehr = jnp.zeros((FP, DP), jnp.float32).at[:F, :D].set(w_ehr)
    if (R, D) != (RP, DP):
        w_proj = jnp.zeros((RP, DP), jnp.float32).at[:R, :D].set(w_proj)
        b_proj = jnp.zeros((1, DP), jnp.float32).at[:, :D].set(b_proj)
        w_cxr = jnp.zeros((C, RP), jnp.float32).at[:, :R].set(w_cxr)
    if (D, C_out) != (DP, CP):
        w_cls_p = (jnp.zeros((2 * DP, CP), jnp.float32)
                   .at[:D, :C_out].set(w_cls[:D])
                   .at[DP:DP + D, :C_out].set(w_cls[D:]))
        b_cls = jnp.zeros((1, CP), jnp.float32).at[:, :C_out].set(b_cls)
    else:
        w_cls_p = w_cls
    if (T, F) != (TP, FP):
        ehr = jnp.pad(ehr, ((0, 0), (0, TP - T), (0, FP - F)))

    if HWP != HW:
        img3 = jnp.pad(img.reshape(B, C, HW), ((0, 0), (0, 0), (0, HWP - HW)))
        imgT = img3.reshape(B, C * HWP).T
    else:
        imgT = img.reshape(B, C * HW).T

    len2 = ehr_len.astype(jnp.int32).reshape(B, 1)

    TB = min(128, _ru(B, SUBLANE))
    BP = _ru(B, TB)
    if BP != B:
        ehr = jnp.pad(ehr, ((0, BP - B), (0, 0), (0, 0)))
        imgT = jnp.pad(imgT, ((0, 0), (0, BP - B)))
        len2 = jnp.pad(len2, ((0, BP - B), (0, 0)))

    grid = (BP // TB,)
    body = functools.partial(_fused_kernel, TP=TP, FP=FP, HW=HW, HWP=HWP,
                             C=C, DP=DP)

    preds_p, proj_p, ehrf_p = pl.pallas_call(
        body,
        out_shape=(
            jax.ShapeDtypeStruct((BP, CP), jnp.float32),
            jax.ShapeDtypeStruct((BP, DP), jnp.float32),
            jax.ShapeDtypeStruct((BP, DP), jnp.float32),
        ),
        grid=grid,
        in_specs=[
            pl.BlockSpec((TB, 1), lambda i: (i, 0)),
            pl.BlockSpec((TB, TP, FP), lambda i: (i, 0, 0)),
            pl.BlockSpec((C * HWP, TB), lambda i: (0, i)),
            pl.BlockSpec((FP, DP), lambda i: (0, 0)),
            pl.BlockSpec((C, RP), lambda i: (0, 0)),
            pl.BlockSpec((RP, DP), lambda i: (0, 0)),
            pl.BlockSpec((1, DP), lambda i: (0, 0)),
            pl.BlockSpec((2 * DP, CP), lambda i: (0, 0)),
            pl.BlockSpec((1, CP), lambda i: (0, 0)),
        ],
        out_specs=[
            pl.BlockSpec((TB, CP), lambda i: (i, 0)),
            pl.BlockSpec((TB, DP), lambda i: (i, 0)),
            pl.BlockSpec((TB, DP), lambda i: (i, 0)),
        ],
        compiler_params=pltpu.CompilerParams(
            dimension_semantics=("parallel",)),
    )(len2, ehr, imgT, w_ehr, w_cxr, w_proj, b_proj, w_cls_p, b_cls)

    return {
        "preds": preds_p[:B, :C_out],
        "ehr_feats": ehrf_p[:B, :D],
        "cxr_feats": proj_p[:B, :D],
    }


# speedup vs baseline: 1.0646x; 1.0646x over previous
import functools

import jax
import jax.numpy as jnp
from jax.experimental import pallas as pl
from jax.experimental.pallas import tpu as pltpu

LANE = 128
SUBLANE = 8


def _ru(x, m):
    return ((x + m - 1) // m) * m


def _fused_kernel(len_ref, ehr_ref, imgT_ref, w_ehr_ref, w_cxr_ref,
                  w_proj_ref, b_proj_ref, w_cls_ref, b_cls_ref,
                  preds_ref, proj_ref, ehrf_ref, *, TP, FP, HW, HWP, C, DP):
    TB = len_ref.shape[0]
    lens = len_ref[...]

    ti = jax.lax.broadcasted_iota(jnp.int32, (TB, TP, 1), 1)
    mask3 = (lens[:, :, None] > ti).astype(jnp.float32)
    acc3 = jnp.zeros((TB, SUBLANE, FP), dtype=jnp.float32)
    for j in range(TP // SUBLANE):
        sl = pl.ds(j * SUBLANE, SUBLANE)
        acc3 = acc3 + ehr_ref[:, sl, :] * mask3[:, j * SUBLANE:(j + 1) * SUBLANE, :]
    acc = jnp.sum(acc3, axis=1)

    inv_len = 1.0 / jnp.maximum(lens.astype(jnp.float32), 1.0)
    ehr_feats = jnp.tanh(
        jnp.dot(acc, w_ehr_ref[...], preferred_element_type=jnp.float32) * inv_len)
    ehrf_ref[...] = ehr_feats

    inv_hw = jnp.float32(1.0 / HW)
    gparts = [jnp.sum(imgT_ref[c * HWP:(c + 1) * HWP, :], axis=0, keepdims=True)
              for c in range(C)]
    g3 = jnp.concatenate(gparts, axis=0) * inv_hw
    cxrT = jax.nn.relu(
        jax.lax.dot_general(w_cxr_ref[...], g3,
                            dimension_numbers=(((0,), (0,)), ((), ())),
                            preferred_element_type=jnp.float32))
    projT = (jax.lax.dot_general(w_proj_ref[...], cxrT,
                                 dimension_numbers=(((0,), (0,)), ((), ())),
                                 preferred_element_type=jnp.float32)
             + jnp.transpose(b_proj_ref[...]))

    proj_ref[...] = jnp.transpose(projT)

    logits = (jnp.dot(ehr_feats, w_cls_ref[0:DP, :], preferred_element_type=jnp.float32)
              + jax.lax.dot_general(projT, w_cls_ref[DP:2 * DP, :],
                                    dimension_numbers=(((0,), (0,)), ((), ())),
                                    preferred_element_type=jnp.float32)
              + b_cls_ref[...])
    preds_ref[...] = jax.nn.sigmoid(logits)


def kernel(ehr, ehr_len, img, w_ehr, w_cxr, w_proj, b_proj, w_cls, b_cls):
    B, T, F = ehr.shape
    C, H, W = img.shape[1], img.shape[2], img.shape[3]
    R, D = w_proj.shape
    C_out = b_cls.shape[1]
    HW = H * W

    FP = _ru(F, LANE)
    DP = _ru(D, LANE)
    RP = _ru(R, LANE)
    CP = _ru(C_out, LANE)
    HWP = _ru(HW, LANE)
    TP = _ru(T, SUBLANE)

    if (F, D) != (FP, DP):
        w_ehr = jnp.zeros((FP, DP), jnp.float32).at[:F, :D].set(w_ehr)
    if (R, D) != (RP, DP):
        w_proj = jnp.zeros((RP, DP), jnp.float32).at[:R, :D].set(w_proj)
        b_proj = jnp.zeros((1, DP), jnp.float32).at[:, :D].set(b_proj)
        w_cxr = jnp.zeros((C, RP), jnp.float32).at[:, :R].set(w_cxr)
    if (D, C_out) != (DP, CP):
        w_cls_p = (jnp.zeros((2 * DP, CP), jnp.float32)
                   .at[:D, :C_out].set(w_cls[:D])
                   .at[DP:DP + D, :C_out].set(w_cls[D:]))
        b_cls = jnp.zeros((1, CP), jnp.float32).at[:, :C_out].set(b_cls)
    else:
        w_cls_p = w_cls
    if (T, F) != (TP, FP):
        ehr = jnp.pad(ehr, ((0, 0), (0, TP - T), (0, FP - F)))

    if HWP != HW:
        img3 = jnp.pad(img.reshape(B, C, HW), ((0, 0), (0, 0), (0, HWP - HW)))
        imgT = img3.reshape(B, C * HWP).T
    else:
        imgT = img.reshape(B, C * HW).T

    len2 = ehr_len.astype(jnp.int32).reshape(B, 1)

    TB = min(256, _ru(B, SUBLANE))
    BP = _ru(B, TB)
    if BP != B:
        ehr = jnp.pad(ehr, ((0, BP - B), (0, 0), (0, 0)))
        imgT = jnp.pad(imgT, ((0, 0), (0, BP - B)))
        len2 = jnp.pad(len2, ((0, BP - B), (0, 0)))

    grid = (BP // TB,)
    body = functools.partial(_fused_kernel, TP=TP, FP=FP, HW=HW, HWP=HWP,
                             C=C, DP=DP)

    preds_p, proj_p, ehrf_p = pl.pallas_call(
        body,
        out_shape=(
            jax.ShapeDtypeStruct((BP, CP), jnp.float32),
            jax.ShapeDtypeStruct((BP, DP), jnp.float32),
            jax.ShapeDtypeStruct((BP, DP), jnp.float32),
        ),
        grid=grid,
        in_specs=[
            pl.BlockSpec((TB, 1), lambda i: (i, 0)),
            pl.BlockSpec((TB, TP, FP), lambda i: (i, 0, 0)),
            pl.BlockSpec((C * HWP, TB), lambda i: (0, i)),
            pl.BlockSpec((FP, DP), lambda i: (0, 0)),
            pl.BlockSpec((C, RP), lambda i: (0, 0)),
            pl.BlockSpec((RP, DP), lambda i: (0, 0)),
            pl.BlockSpec((1, DP), lambda i: (0, 0)),
            pl.BlockSpec((2 * DP, CP), lambda i: (0, 0)),
            pl.BlockSpec((1, CP), lambda i: (0, 0)),
        ],
        out_specs=[
            pl.BlockSpec((TB, CP), lambda i: (i, 0)),
            pl.BlockSpec((TB, DP), lambda i: (i, 0)),
            pl.BlockSpec((TB, DP), lambda i: (i, 0)),
        ],
        compiler_params=pltpu.CompilerParams(
            dimension_semantics=("parallel",)),
    )(len2, ehr, imgT, w_ehr, w_cxr, w_proj, b_proj, w_cls_p, b_cls)

    return {
        "preds": preds_p[:B, :C_out],
        "ehr_feats": ehrf_p[:B, :D],
        "cxr_feats": proj_p[:B, :D],
    }


# speedup vs baseline: 1.1008x; 1.0340x over previous
import functools

import jax
import jax.numpy as jnp
from jax.experimental import pallas as pl
from jax.experimental.pallas import tpu as pltpu

LANE = 128
SUBLANE = 8


def _ru(x, m):
    return ((x + m - 1) // m) * m


def _fused_kernel(len_ref, ehr_ref, imgT_ref, w_ehr_ref, w_cxr_ref,
                  w_proj_ref, b_proj_ref, w_cls_ref, b_cls_ref,
                  preds_ref, proj_ref, ehrf_ref, *, TP, FP, HW, HWP, C, DP):
    TB = len_ref.shape[1]
    lens = jnp.transpose(len_ref[...])

    ti = jax.lax.broadcasted_iota(jnp.int32, (TB, TP, 1), 1)
    mask3 = (lens[:, :, None] > ti).astype(jnp.float32)
    acc3 = jnp.zeros((TB, SUBLANE, FP), dtype=jnp.float32)
    for j in range(TP // SUBLANE):
        sl = pl.ds(j * SUBLANE, SUBLANE)
        acc3 = acc3 + ehr_ref[:, sl, :] * mask3[:, j * SUBLANE:(j + 1) * SUBLANE, :]
    acc = jnp.sum(acc3, axis=1)

    inv_len = 1.0 / jnp.maximum(lens.astype(jnp.float32), 1.0)
    ehr_feats = jnp.tanh(
        jnp.dot(acc, w_ehr_ref[...], preferred_element_type=jnp.float32) * inv_len)
    ehrf_ref[...] = ehr_feats

    inv_hw = jnp.float32(1.0 / HW)
    gparts = [jnp.sum(imgT_ref[c * HWP:(c + 1) * HWP, :], axis=0, keepdims=True)
              for c in range(C)]
    g3 = jnp.concatenate(gparts, axis=0) * inv_hw
    cxrT = jax.nn.relu(
        jax.lax.dot_general(w_cxr_ref[...], g3,
                            dimension_numbers=(((0,), (0,)), ((), ())),
                            preferred_element_type=jnp.float32))
    projT = (jax.lax.dot_general(w_proj_ref[...], cxrT,
                                 dimension_numbers=(((0,), (0,)), ((), ())),
                                 preferred_element_type=jnp.float32)
             + jnp.transpose(b_proj_ref[...]))

    proj_ref[...] = jnp.transpose(projT)

    logits = (jnp.dot(ehr_feats, w_cls_ref[0:DP, :], preferred_element_type=jnp.float32)
              + jax.lax.dot_general(projT, w_cls_ref[DP:2 * DP, :],
                                    dimension_numbers=(((0,), (0,)), ((), ())),
                                    preferred_element_type=jnp.float32)
              + b_cls_ref[...])
    preds_ref[...] = jax.nn.sigmoid(logits)


def kernel(ehr, ehr_len, img, w_ehr, w_cxr, w_proj, b_proj, w_cls, b_cls):
    B, T, F = ehr.shape
    C, H, W = img.shape[1], img.shape[2], img.shape[3]
    R, D = w_proj.shape
    C_out = b_cls.shape[1]
    HW = H * W

    FP = _ru(F, LANE)
    DP = _ru(D, LANE)
    RP = _ru(R, LANE)
    CP = _ru(C_out, LANE)
    HWP = _ru(HW, LANE)
    TP = _ru(T, SUBLANE)

    if (F, D) != (FP, DP):
        w_ehr = jnp.zeros((FP, DP), jnp.float32).at[:F, :D].set(w_ehr)
    if (R, D) != (RP, DP):
        w_proj = jnp.zeros((RP, DP), jnp.float32).at[:R, :D].set(w_proj)
        b_proj = jnp.zeros((1, DP), jnp.float32).at[:, :D].set(b_proj)
        w_cxr = jnp.zeros((C, RP), jnp.float32).at[:, :R].set(w_cxr)
    if (D, C_out) != (DP, CP):
        w_cls_p = (jnp.zeros((2 * DP, CP), jnp.float32)
                   .at[:D, :C_out].set(w_cls[:D])
                   .at[DP:DP + D, :C_out].set(w_cls[D:]))
        b_cls = jnp.zeros((1, CP), jnp.float32).at[:, :C_out].set(b_cls)
    else:
        w_cls_p = w_cls
    if (T, F) != (TP, FP):
        ehr = jnp.pad(ehr, ((0, 0), (0, TP - T), (0, FP - F)))

    if HWP != HW:
        img3 = jnp.pad(img.reshape(B, C, HW), ((0, 0), (0, 0), (0, HWP - HW)))
        imgT = img3.reshape(B, C * HWP).T
    else:
        imgT = img.reshape(B, C * HW).T

    len2 = ehr_len.astype(jnp.int32).reshape(1, B)

    TB = min(256, _ru(B, SUBLANE))
    BP = _ru(B, TB)
    if BP != B:
        ehr = jnp.pad(ehr, ((0, BP - B), (0, 0), (0, 0)))
        imgT = jnp.pad(imgT, ((0, 0), (0, BP - B)))
        len2 = jnp.pad(len2, ((0, 0), (0, BP - B)))

    grid = (BP // TB,)
    body = functools.partial(_fused_kernel, TP=TP, FP=FP, HW=HW, HWP=HWP,
                             C=C, DP=DP)

    preds_p, proj_p, ehrf_p = pl.pallas_call(
        body,
        out_shape=(
            jax.ShapeDtypeStruct((BP, CP), jnp.float32),
            jax.ShapeDtypeStruct((BP, DP), jnp.float32),
            jax.ShapeDtypeStruct((BP, DP), jnp.float32),
        ),
        grid=grid,
        in_specs=[
            pl.BlockSpec((1, TB), lambda i: (0, i)),
            pl.BlockSpec((TB, TP, FP), lambda i: (i, 0, 0)),
            pl.BlockSpec((C * HWP, TB), lambda i: (0, i)),
            pl.BlockSpec((FP, DP), lambda i: (0, 0)),
            pl.BlockSpec((C, RP), lambda i: (0, 0)),
            pl.BlockSpec((RP, DP), lambda i: (0, 0)),
            pl.BlockSpec((1, DP), lambda i: (0, 0)),
            pl.BlockSpec((2 * DP, CP), lambda i: (0, 0)),
            pl.BlockSpec((1, CP), lambda i: (0, 0)),
        ],
        out_specs=[
            pl.BlockSpec((TB, CP), lambda i: (i, 0)),
            pl.BlockSpec((TB, DP), lambda i: (i, 0)),
            pl.BlockSpec((TB, DP), lambda i: (i, 0)),
        ],
        compiler_params=pltpu.CompilerParams(
            dimension_semantics=("parallel",)),
    )(len2, ehr, imgT, w_ehr, w_cxr, w_proj, b_proj, w_cls_p, b_cls)

    return {
        "preds": preds_p[:B, :C_out],
        "ehr_feats": ehrf_p[:B, :D],
        "cxr_feats": proj_p[:B, :D],
    }
